# baseline (speedup 1.0000x reference)
import functools

import numpy as np
import jax
import jax.numpy as jnp
from jax import lax
from jax.experimental import pallas as pl
from jax.experimental.pallas import tpu as pltpu
from jax.experimental.pallas import tpu_sc as plsc

_K = 2
_S = 16000
_SUBW = 25000
_NSC = 2

_plan_cache = {}


def _tf2x32(k1, k2, x0, x1):
    rot_a = (13, 15, 26, 6)
    rot_b = (17, 29, 16, 24)
    ks = [np.uint32(k1), np.uint32(k2),
          np.uint32(k1) ^ np.uint32(k2) ^ np.uint32(0x1BD11BDA)]
    x0 = (x0 + ks[0]).astype(np.uint32)
    x1 = (x1 + ks[1]).astype(np.uint32)

    def rnd(x0, x1, r):
        x0 = (x0 + x1).astype(np.uint32)
        x1 = ((x1 << np.uint32(r)) | (x1 >> np.uint32(32 - r))).astype(np.uint32)
        return x0, x1 ^ x0

    rots = (rot_a, rot_b, rot_a, rot_b, rot_a)
    for g in range(5):
        for r in rots[g]:
            x0, x1 = rnd(x0, x1, r)
        x0 = (x0 + ks[(g + 1) % 3]).astype(np.uint32)
        x1 = (x1 + ks[(g + 2) % 3] + np.uint32(g + 1)).astype(np.uint32)
    return x0, x1


def _np_permutation(seed, n):
    key = (np.uint32(seed >> 32), np.uint32(seed & 0xFFFFFFFF))
    num_rounds = int(np.ceil(3 * np.log(max(1, n))
                             / np.log(np.iinfo(np.uint32).max)))
    x = np.arange(n, dtype=np.int64)
    for _ in range(num_rounds):
        b1, b2 = _tf2x32(key[0], key[1],
                         np.zeros(2, np.uint32), np.arange(2, dtype=np.uint32))
        key, sub = (b1[0], b2[0]), (b1[1], b2[1])
        s1, s2 = _tf2x32(sub[0], sub[1],
                         np.zeros(n, np.uint32), np.arange(n, dtype=np.uint32))
        x = x[np.argsort(s1 ^ s2, kind="stable")]
    return x


def _host_plan(n_tab, n_out):
    key = (n_tab, n_out)
    if key not in _plan_cache:
        g = (_np_permutation(42, n_out) // _K).astype(np.int64)
        n_ch = n_out // _S
        ch_sc = n_ch // _NSC
        ns = n_tab // _SUBW
        i = np.arange(n_out, dtype=np.int64)
        d = i // _S
        c = (d >= ch_sc).astype(np.int64)
        dl = d % ch_sc
        s = g // _SUBW
        toff = (g % _SUBW).astype(np.int32)

        order = np.lexsort((i, dl, s, c))
        gid = (c * ns + s) * ch_sc + dl
        gid_sorted = gid[order]
        change = np.r_[True, gid_sorted[1:] != gid_sorted[:-1]]
        gstart = np.maximum.accumulate(np.where(change, np.arange(n_out), 0))
        rank_sorted = np.arange(n_out) - gstart
        counts = np.bincount(gid, minlength=_NSC * ns * ch_sc)
        r_run = int(((counts.max() + 7) // 8) * 8)

        stageplan = np.zeros((_NSC * ns, ch_sc * r_run), np.int32)
        stageplan[(c * ns + s)[order], dl[order] * r_run + rank_sorted] = \
            toff[order]
        rank = np.empty(n_out, dtype=np.int64)
        rank[order] = rank_sorted
        p2 = (s * r_run + rank).astype(np.int32).reshape(n_ch, _S)
        _plan_cache[key] = (stageplan, p2, r_run)
    return _plan_cache[key]


@functools.lru_cache(maxsize=None)
def _build_shuffle(n_tab, n_out, r_run):
    info = plsc.get_sparse_core_info()
    nc, nsub = info.num_cores, info.num_subcores
    assert nc == _NSC and nsub == 16
    n_ch = n_out // _S
    ch_sc = n_ch // _NSC
    ns = n_tab // _SUBW
    assert n_out % _S == 0 and n_ch % _NSC == 0 and n_tab % _SUBW == 0
    assert ns % nsub == 0 and (2 * ch_sc) % nsub == 0
    si_per_tile = ns // nsub
    jobs_per_tile = 2 * ch_sc // nsub
    stage_w = ch_sc * r_run
    reg_w = ns * r_run
    n_inter = _NSC * ch_sc * ns * r_run

    mesh = plsc.VectorSubcoreMesh(core_axis_name="c", subcore_axis_name="s")

    @functools.partial(
        pl.kernel,
        mesh=mesh,
        compiler_params=pltpu.CompilerParams(needs_layout_passes=False),
        out_type=[
            jax.ShapeDtypeStruct((n_out,), jnp.int32),
            jax.ShapeDtypeStruct((n_out,), jnp.int32),
            jax.ShapeDtypeStruct((n_inter,), jnp.int32),
        ],
        scratch_types=[
            pltpu.VMEM((_SUBW,), jnp.int32),
            pltpu.VMEM((stage_w,), jnp.int32),
            pltpu.VMEM((stage_w,), jnp.int32),
            pltpu.SemaphoreType.DMA,
            pltpu.SemaphoreType.DMA,
            pltpu.SemaphoreType.DMA,
        ],
    )
    def shuffle_kernel(dst_tab, src_tab, stageplan, p2, out_dst, out_src,
                       inter, in_v, plan_v, stage_v, isem, psem, osem):
        c = lax.axis_index("c")
        tid = lax.axis_index("s")
        half_iota = lax.shift_right_logical(lax.iota(jnp.int32, 16), 1)

        for si in range(si_per_tile):
            s = tid * si_per_tile + si
            h1 = pltpu.async_copy(
                dst_tab.at[pl.ds(s * _SUBW, _SUBW)], in_v, isem)
            h2 = pltpu.async_copy(stageplan.at[c * ns + s], plan_v, isem)
            h1.wait()
            h2.wait()

            def stage_one(j, carry):
                in_v_idx = plan_v[pl.ds(j * 16, 16)]
                stage_v[pl.ds(j * 16, 16)] = plsc.load_gather(
                    in_v, [in_v_idx])
                return carry

            lax.fori_loop(0, stage_w // 16, stage_one, 0, unroll=8)

            def put_run(dl, carry):
                base = ((c * ch_sc + dl) * ns + s) * r_run
                pltpu.async_copy(
                    stage_v.at[pl.ds(dl * r_run, r_run)],
                    inter.at[pl.ds(base, r_run)],
                    psem,
                )
                return carry

            lax.fori_loop(0, ch_sc, put_run, 0)
            pltpu.make_async_copy(
                dst_tab.at[pl.ds(0, stage_w)], stage_v, psem).wait()

        plsc.subcore_barrier()

        def job(jl, carry):
            jid = tid * jobs_per_tile + jl

            @pl.when(jid < ch_sc)
            def _():
                dl = jid
                h1 = pltpu.async_copy(
                    inter.at[pl.ds((c * ch_sc + dl) * reg_w, reg_w)],
                    stage_v.at[pl.ds(0, reg_w)], isem)
                h2 = pltpu.async_copy(
                    p2.at[c * ch_sc + dl], plan_v.at[pl.ds(0, _S)], isem)
                h1.wait()
                h2.wait()

                @pl.when(jl >= 1)
                def _():
                    pltpu.make_async_copy(
                        in_v.at[pl.ds(0, _S)],
                        out_dst.at[pl.ds(0, _S)], osem).wait()

                def unperm_one(j, carry2):
                    reg_idx = plan_v[pl.ds(j * 16, 16)]
                    in_v[pl.ds(j * 16, 16)] = plsc.load_gather(
                        stage_v, [reg_idx])
                    return carry2

                lax.fori_loop(0, _S // 16, unperm_one, 0, unroll=8)
                pltpu.async_copy(
                    in_v.at[pl.ds(0, _S)],
                    out_dst.at[pl.ds((c * ch_sc + dl) * _S, _S)], osem)

            @pl.when(jid >= ch_sc)
            def _():
                gc = c * ch_sc + (jid - ch_sc)
                h = pltpu.async_copy(
                    src_tab.at[pl.ds(gc * (_S // _K), _S // _K)],
                    plan_v.at[pl.ds(0, _S // _K)], isem)
                h.wait()

                @pl.when(jl >= 1)
                def _():
                    pltpu.make_async_copy(
                        in_v.at[pl.ds(0, _S)],
                        out_dst.at[pl.ds(0, _S)], osem).wait()

                def double_one(j, carry2):
                    in_v[pl.ds(j * 16, 16)] = plsc.load_gather(
                        plan_v, [j * 8 + half_iota])
                    return carry2

                lax.fori_loop(0, _S // 16, double_one, 0, unroll=8)
                pltpu.async_copy(
                    in_v.at[pl.ds(0, _S)],
                    out_src.at[pl.ds(gc * _S, _S)], osem)

            return carry

        lax.fori_loop(0, jobs_per_tile, job, 0)
        pltpu.make_async_copy(
            in_v.at[pl.ds(0, _S)], out_dst.at[pl.ds(0, _S)], osem).wait()

    return shuffle_kernel


def kernel(edge_dst, edge_src, node_feature):
    n_tab = edge_dst.shape[0]
    n_out = n_tab * _K
    sp, p2, r_run = _host_plan(n_tab, n_out)
    shuffle = _build_shuffle(n_tab, n_out, r_run)
    out_dst, out_src, _ = shuffle(
        edge_dst.astype(jnp.int32),
        edge_src.astype(jnp.int32),
        jnp.asarray(sp),
        jnp.asarray(p2),
    )
    dt = edge_dst.dtype
    return out_dst.astype(dt), out_src.astype(dt), node_feature


# speedup vs baseline: 1.5061x; 1.5061x over previous
import functools

import numpy as np
import jax
import jax.numpy as jnp
from jax import lax
from jax.experimental import pallas as pl
from jax.experimental.pallas import tpu as pltpu
from jax.experimental.pallas import tpu_sc as plsc

_K = 2
_ROW = 128
_ROWS = 125
_CHUNK = _ROW * _ROWS
_HALF = _CHUNK // _K
_NB = 2

_plan_cache = {}


def _tf2x32(k1, k2, x0, x1):
    rot_a = (13, 15, 26, 6)
    rot_b = (17, 29, 16, 24)
    ks = [np.uint32(k1), np.uint32(k2),
          np.uint32(k1) ^ np.uint32(k2) ^ np.uint32(0x1BD11BDA)]
    x0 = (x0 + ks[0]).astype(np.uint32)
    x1 = (x1 + ks[1]).astype(np.uint32)

    def rnd(x0, x1, r):
        x0 = (x0 + x1).astype(np.uint32)
        x1 = ((x1 << np.uint32(r)) | (x1 >> np.uint32(32 - r))).astype(np.uint32)
        return x0, x1 ^ x0

    rots = (rot_a, rot_b, rot_a, rot_b, rot_a)
    for g in range(5):
        for r in rots[g]:
            x0, x1 = rnd(x0, x1, r)
        x0 = (x0 + ks[(g + 1) % 3]).astype(np.uint32)
        x1 = (x1 + ks[(g + 2) % 3] + np.uint32(g + 1)).astype(np.uint32)
    return x0, x1


def _np_permutation(seed, n):
    key = (np.uint32(seed >> 32), np.uint32(seed & 0xFFFFFFFF))
    num_rounds = int(np.ceil(3 * np.log(max(1, n))
                             / np.log(np.iinfo(np.uint32).max)))
    x = np.arange(n, dtype=np.int64)
    for _ in range(num_rounds):
        b1, b2 = _tf2x32(key[0], key[1],
                         np.zeros(2, np.uint32), np.arange(2, dtype=np.uint32))
        key, sub = (b1[0], b2[0]), (b1[1], b2[1])
        s1, s2 = _tf2x32(sub[0], sub[1],
                         np.zeros(n, np.uint32), np.arange(n, dtype=np.uint32))
        x = x[np.argsort(s1 ^ s2, kind="stable")]
    return x


def _host_plan(n_out):
    if n_out not in _plan_cache:
        perm = _np_permutation(42, n_out)
        g = (perm // _K).astype(np.int32).reshape(-1, _ROWS, _ROW)
        _plan_cache[n_out] = g
    return _plan_cache[n_out]


@functools.lru_cache(maxsize=None)
def _build_gather(n_out):
    info = plsc.get_sparse_core_info()
    nc, ns = info.num_cores, info.num_subcores
    nw = nc * ns
    n_chunks = n_out // _CHUNK
    assert n_out % _CHUNK == 0
    n_jobs = 2 * n_chunks
    assert n_jobs % nw == 0
    steps = n_jobs // nw

    mesh = plsc.VectorSubcoreMesh(core_axis_name="c", subcore_axis_name="s")

    @functools.partial(
        pl.kernel,
        mesh=mesh,
        compiler_params=pltpu.CompilerParams(needs_layout_passes=False),
        out_type=[
            jax.ShapeDtypeStruct((n_out,), jnp.int32),
            jax.ShapeDtypeStruct((n_out,), jnp.int32),
        ],
        scratch_types=[
            pltpu.VMEM((_ROWS, _ROW), jnp.int32),
            pltpu.VMEM((_ROWS, _ROW), jnp.int32),
            pltpu.VMEM((_CHUNK,), jnp.int32),
            pltpu.VMEM((_CHUNK,), jnp.int32),
            pltpu.VMEM((_HALF,), jnp.int32),
            pltpu.VMEM((_HALF,), jnp.int32),
            pltpu.SemaphoreType.DMA,
            pltpu.SemaphoreType.DMA,
            pltpu.SemaphoreType.DMA,
            pltpu.SemaphoreType.DMA,
            pltpu.SemaphoreType.DMA,
            pltpu.SemaphoreType.DMA,
        ],
    )
    def gather_kernel(dst_tab, src_tab, gidx3, out_dst, out_src,
                      idx_a, idx_b, buf_a, buf_b, sbuf_a, sbuf_b,
                      isem_a, isem_b, gsem_a, gsem_b, osem_a, osem_b):
        wid = lax.axis_index("s") * nc + lax.axis_index("c")
        idx_v = (idx_a, idx_b)
        buf_v = (buf_a, buf_b)
        sbuf_v = (sbuf_a, sbuf_b)
        isem = (isem_a, isem_b)
        gsem = (gsem_a, gsem_b)
        osem = (osem_a, osem_b)

        def for_job(q, dst_fn, src_fn):
            @pl.when(q < n_chunks)
            def _():
                dst_fn(q)

            @pl.when(q >= n_chunks)
            def _():
                src_fn(q - n_chunks)

        def prefetch(q, b):
            for_job(
                q,
                lambda c: pltpu.async_copy(gidx3.at[c], idx_v[b], isem[b]),
                lambda c: pltpu.async_copy(
                    src_tab.at[pl.ds(c * _HALF, _HALF)], sbuf_v[b], isem[b]),
            )

        def wait_prefetch(q, b):
            for_job(
                q,
                lambda c: pltpu.make_async_copy(
                    gidx3.at[0], idx_v[b], isem[b]).wait(),
                lambda c: pltpu.make_async_copy(
                    src_tab.at[pl.ds(0, _HALF)], sbuf_v[b], isem[b]).wait(),
            )

        def process(q, b):
            def dst_fn(c):
                def one(j, carry):
                    pltpu.async_copy(
                        dst_tab.at[idx_v[b].at[j]],
                        buf_v[b].at[pl.ds(j * _ROW, _ROW)],
                        gsem[b],
                    )
                    return carry
                lax.fori_loop(0, _ROWS, one, 0)
                pltpu.make_async_copy(
                    dst_tab.at[pl.ds(0, _CHUNK)], buf_v[b], gsem[b]
                ).wait()

            def src_fn(c):
                half_iota = lax.shift_right_logical(
                    lax.iota(jnp.int32, 16), 1)

                def one(j, carry):
                    v = plsc.load_gather(sbuf_v[b], [j * 8 + half_iota])
                    buf_v[b][pl.ds(j * 16, 16)] = v
                    return carry

                lax.fori_loop(0, _CHUNK // 16, one, 0, unroll=8)

            for_job(q, dst_fn, src_fn)

        def writeback(q, b):
            for_job(
                q,
                lambda c: pltpu.async_copy(
                    buf_v[b], out_dst.at[pl.ds(c * _CHUNK, _CHUNK)], osem[b]),
                lambda c: pltpu.async_copy(
                    buf_v[b], out_src.at[pl.ds(c * _CHUNK, _CHUNK)], osem[b]),
            )

        def wait_out(b):
            pltpu.make_async_copy(
                buf_v[b], out_dst.at[pl.ds(0, _CHUNK)], osem[b]
            ).wait()

        for b in range(_NB):
            prefetch(wid + b * nw, b)

        def step(k2, carry):
            for b in range(_NB):
                k = k2 * _NB + b
                q = wid + k * nw
                wait_prefetch(q, b)

                @pl.when(k >= _NB)
                def _():
                    wait_out(b)

                process(q, b)
                writeback(q, b)

                @pl.when(k + _NB < steps)
                def _():
                    prefetch(wid + (k + _NB) * nw, b)

            return carry

        assert steps % _NB == 1
        lax.fori_loop(0, steps // _NB, step, 0)

        k = steps - 1
        q = wid + k * nw
        wait_prefetch(q, 0)
        wait_out(0)
        process(q, 0)
        writeback(q, 0)
        wait_out(1)
        wait_out(0)

    return gather_kernel


def kernel(edge_dst, edge_src, node_feature):
    n_out = edge_dst.shape[0] * _K
    g3 = _host_plan(n_out)
    gather = _build_gather(n_out)
    out_dst, out_src = gather(
        edge_dst.astype(jnp.int32),
        edge_src.astype(jnp.int32),
        jnp.asarray(g3),
    )
    dt = edge_dst.dtype
    return out_dst.astype(dt), out_src.astype(dt), node_feature
